# baseline (speedup 1.0000x reference)
import functools

import jax
import jax.numpy as jnp
from jax import lax
from jax.experimental import pallas as pl
from jax.experimental.pallas import tpu as pltpu
from jax.experimental.pallas import tpu_sc as plsc

D = 128
LANES = 16
N_DST = 50000
N_CHUNKS = 6
S_CHUNK = 8448
N_PAD = N_CHUNKS * S_CHUNK
ACC_R = 8576
DEG_R = 8704
DEG_OUT = N_CHUNKS * DEG_R
BATCH = 128
SECT = 4000


@functools.lru_cache(maxsize=None)
def _build_sc_agg(n_src, n_edges, with_deg):
    stripe = n_edges // 16
    n_sect = stripe // SECT
    cap = SECT + 160
    dump = cap - LANES
    mesh = plsc.VectorSubcoreMesh(core_axis_name="c", subcore_axis_name="s",
                                  num_cores=2, num_subcores=16)

    out_type = [jax.ShapeDtypeStruct((N_PAD, D), jnp.float32)]
    if with_deg:
        out_type.append(jax.ShapeDtypeStruct((DEG_OUT,), jnp.float32))

    scratch = [
        pltpu.VMEM((stripe,), jnp.int32),
        pltpu.VMEM((stripe,), jnp.int32),
        pltpu.VMEM((cap,), jnp.int32),
        pltpu.VMEM((cap,), jnp.int32),
        pltpu.VMEM((1, BATCH // 2), jnp.int32),
        pltpu.VMEM((1, BATCH // 2), jnp.int32),
        pltpu.VMEM((BATCH, D), jnp.float32),
        pltpu.VMEM_SHARED((ACC_R, D), jnp.float32),
        pltpu.SemaphoreType.DMA,
        pltpu.SemaphoreType.DMA,
        pltpu.SemaphoreType.DMA,
        pltpu.SemaphoreType.DMA,
    ]
    if with_deg:
        scratch += [
            pltpu.VMEM((BATCH,), jnp.float32),
            pltpu.VMEM((544,), jnp.float32),
            pltpu.VMEM((544,), jnp.float32),
            pltpu.VMEM_SHARED((DEG_R,), jnp.float32),
        ]

    def body(x_hbm, src_hbm, dst_hbm, out_hbm, *rest):
        if with_deg:
            (deg_hbm, sstage, dstage, gflat, lflat, curla, curlb, rows,
             acc, gsa, gsb, ssa, ssb, ones_v, zdeg, dstg, dacc) = rest
        else:
            (sstage, dstage, gflat, lflat, curla, curlb, rows,
             acc, gsa, gsb, ssa, ssb) = rest
        half = BATCH // 2
        cid = lax.axis_index("c")
        sid = lax.axis_index("s")
        zv = jnp.zeros((LANES,), jnp.float32)
        lane = lax.iota(jnp.int32, LANES)
        zvi = jnp.zeros((LANES,), jnp.int32)
        tvi = jnp.full((LANES,), S_CHUNK, jnp.int32)

        if with_deg:
            ov = jnp.ones((LANES,), jnp.float32)
            for k in range(BATCH // LANES):
                ones_v[pl.ds(k * LANES, LANES)] = ov

            def zd(i, _):
                zdeg[pl.ds(i * LANES, LANES)] = zv
                return 0
            lax.fori_loop(0, 544 // LANES, zd, 0)

        def zero_rows(i, _):
            rows[i // 8, pl.ds((i % 8) * LANES, LANES)] = zv
            return 0

        pltpu.sync_copy(src_hbm.at[pl.ds(sid * stripe, stripe)], sstage)
        pltpu.sync_copy(dst_hbm.at[pl.ds(sid * stripe, stripe)], dstage)

        def drain_scatters():
            pltpu.make_async_copy(rows.at[pl.ds(0, half)],
                                  acc.at[curla.at[0]], ssa).wait()
            pltpu.make_async_copy(rows.at[pl.ds(half, half)],
                                  acc.at[curlb.at[0]], ssb).wait()
            if with_deg:
                pltpu.make_async_copy(ones_v.at[pl.ds(0, half)],
                                      dacc.at[curla.at[0]], ssa).wait()
                pltpu.make_async_copy(ones_v.at[pl.ds(0, half)],
                                      dacc.at[curlb.at[0]], ssb).wait()

        def issue_scatters():
            pltpu.async_copy(rows.at[pl.ds(0, half)], acc.at[curla.at[0]],
                             ssa, add=True)
            pltpu.async_copy(rows.at[pl.ds(half, half)], acc.at[curlb.at[0]],
                             ssb, add=True)
            if with_deg:
                pltpu.async_copy(ones_v.at[pl.ds(0, half)],
                                 dacc.at[curla.at[0]], ssa, add=True)
                pltpu.async_copy(ones_v.at[pl.ds(0, half)],
                                 dacc.at[curlb.at[0]], ssb, add=True)

        def fire(j, _):
            drain_scatters()
            ga = pltpu.async_copy(
                x_hbm.at[gflat.at[pl.ds(j * BATCH, half)]],
                rows.at[pl.ds(0, half)], gsa)
            gb = pltpu.async_copy(
                x_hbm.at[gflat.at[pl.ds(j * BATCH + half, half)]],
                rows.at[pl.ds(half, half)], gsb)
            for k in range(half // LANES):
                curla[0, pl.ds(k * LANES, LANES)] = lflat[pl.ds(j * BATCH + k * LANES, LANES)]
                curlb[0, pl.ds(k * LANES, LANES)] = lflat[pl.ds(j * BATCH + half + k * LANES, LANES)]
            ga.wait()
            gb.wait()
            issue_scatters()
            return 0

        for c_local in range(N_CHUNKS // 2):
            chunk = cid * (N_CHUNKS // 2) + c_local
            lo = chunk * S_CHUNK

            lax.fori_loop(0, BATCH * 8, zero_rows, 0)
            r0 = sid * (ACC_R // 16)
            for off, ln in ((0, 128), (128, 128), (256, 128), (384, 128),
                            (512, 24)):
                pltpu.sync_copy(rows.at[pl.ds(0, ln)], acc.at[pl.ds(r0 + off, ln)])
            if with_deg:
                pltpu.sync_copy(zdeg, dacc.at[pl.ds(sid * 544, 544)])
            plsc.subcore_barrier()
            for k in range(half // LANES):
                curla[0, pl.ds(k * LANES, LANES)] = tvi
                curlb[0, pl.ds(k * LANES, LANES)] = tvi
            issue_scatters()

            def section(s, f):
                base = s * SECT

                def comp(i, fc):
                    o = base + i * (5 * LANES)
                    pos_base = fc
                    for u in range(5):
                        sv = sstage[pl.ds(o + u * LANES, LANES)]
                        dv = dstage[pl.ds(o + u * LANES, LANES)]
                        dl = dv - lo
                        m = (dl >= 0) & (dl < S_CHUNK)
                        mi = m.astype(jnp.int32)
                        ex = jnp.cumsum(mi) - mi
                        pos = jnp.where(m, pos_base + ex, dump + lane)
                        plsc.store_scatter(gflat, [pos], sv)
                        plsc.store_scatter(lflat, [pos], dl)
                        pos_base = pos_base + jnp.sum(mi)
                    return pos_base
                f = lax.fori_loop(0, SECT // (5 * LANES), comp, f)
                nbf = f // BATCH
                lax.fori_loop(0, nbf, fire, 0)
                for k in range(BATCH // LANES):
                    gv = gflat[pl.ds(nbf * BATCH + k * LANES, LANES)]
                    lv = lflat[pl.ds(nbf * BATCH + k * LANES, LANES)]
                    gflat[pl.ds(k * LANES, LANES)] = gv
                    lflat[pl.ds(k * LANES, LANES)] = lv
                return f - nbf * BATCH
            f = lax.fori_loop(0, n_sect, section, jnp.int32(0))

            def padb(i, _):
                off = f + i * LANES
                gflat[pl.ds(off, LANES)] = zvi
                lflat[pl.ds(off, LANES)] = tvi
                return 0
            lax.fori_loop(0, (BATCH - f + LANES - 1) // LANES, padb, 0)
            lax.fori_loop(0, (f + BATCH - 1) // BATCH, fire, 0)
            drain_scatters()
            plsc.subcore_barrier()

            e0 = sid * (S_CHUNK // 16)
            pltpu.sync_copy(acc.at[pl.ds(e0, S_CHUNK // 16)],
                            out_hbm.at[pl.ds(lo + e0, S_CHUNK // 16)])
            if with_deg:
                pltpu.sync_copy(dacc.at[pl.ds(sid * 544, 544)], dstg)
                pltpu.sync_copy(dstg,
                                deg_hbm.at[pl.ds(chunk * DEG_R + sid * 544, 544)])
            plsc.subcore_barrier()

    return pl.kernel(
        body, out_type=out_type, mesh=mesh, scratch_types=scratch,
        compiler_params=pltpu.CompilerParams(needs_layout_passes=False))


def _sc_agg(x, src, dst, with_deg):
    fn = _build_sc_agg(x.shape[0], src.shape[0], with_deg)
    out = fn(x, src, dst)
    return out if with_deg else out[0]


def _dense(x, agg, deg, Ws, Wn, b, relu):
    n = x.shape[0]
    blk = 400

    def body(x_ref, a_ref, d_ref, ws_ref, wn_ref, b_ref, o_ref):
        inv = 1.0 / jnp.maximum(d_ref[...], 1.0)
        h = a_ref[...] * inv
        acc = jnp.dot(x_ref[...], ws_ref[...], preferred_element_type=jnp.float32)
        acc = acc + jnp.dot(h, wn_ref[...], preferred_element_type=jnp.float32)
        acc = acc + b_ref[...]
        if relu:
            acc = jnp.maximum(acc, 0.0)
        o_ref[...] = acc

    return pl.pallas_call(
        body,
        grid=(n // blk,),
        in_specs=[
            pl.BlockSpec((blk, D), lambda i: (i, 0)),
            pl.BlockSpec((blk, D), lambda i: (i, 0)),
            pl.BlockSpec((blk, 1), lambda i: (i, 0)),
            pl.BlockSpec((D, D), lambda i: (0, 0)),
            pl.BlockSpec((D, D), lambda i: (0, 0)),
            pl.BlockSpec((1, D), lambda i: (0, 0)),
        ],
        out_specs=pl.BlockSpec((blk, D), lambda i: (i, 0)),
        out_shape=jax.ShapeDtypeStruct((n, D), jnp.float32),
    )(x, agg, deg, Ws, Wn, b.reshape(1, D))


def kernel(x_user, x_item, edge_index_clicks, edge_index_clicked_by,
           Wn0_c, Ws0_c, b0_c, Wn0_cb, Ws0_cb, b0_cb,
           Wn1_c, Ws1_c, b1_c, Wn1_cb, Ws1_cb, b1_cb):
    sc = edge_index_clicks[0].astype(jnp.int32)
    dc = edge_index_clicks[1].astype(jnp.int32)
    scb = edge_index_clicked_by[0].astype(jnp.int32)
    dcb = edge_index_clicked_by[1].astype(jnp.int32)

    agg0_c, deg_c_raw = _sc_agg(x_user, sc, dc, True)
    agg0_cb, deg_cb_raw = _sc_agg(x_item, scb, dcb, True)
    deg_c = deg_c_raw.reshape(N_CHUNKS, DEG_R)[:, :S_CHUNK].reshape(N_PAD, 1)
    deg_cb = deg_cb_raw.reshape(N_CHUNKS, DEG_R)[:, :S_CHUNK].reshape(N_PAD, 1)

    h_item = _dense(x_item, agg0_c, deg_c, Ws0_c, Wn0_c, b0_c, True)
    h_user = _dense(x_user, agg0_cb, deg_cb, Ws0_cb, Wn0_cb, b0_cb, True)

    agg1_c = _sc_agg(h_user, sc, dc, False)
    agg1_cb = _sc_agg(h_item, scb, dcb, False)

    out_item = _dense(h_item, agg1_c, deg_c, Ws1_c, Wn1_c, b1_c, False)
    out_user = _dense(h_user, agg1_cb, deg_cb, Ws1_cb, Wn1_cb, b1_cb, False)
    return (out_user, out_item)


# speedup vs baseline: 1.0167x; 1.0167x over previous
import functools

import jax
import jax.numpy as jnp
from jax import lax
from jax.experimental import pallas as pl
from jax.experimental.pallas import tpu as pltpu
from jax.experimental.pallas import tpu_sc as plsc

D = 128
LANES = 16
N_DST = 50000
N_CHUNKS = 6
S_CHUNK = 8448
N_PAD = N_CHUNKS * S_CHUNK
ACC_R = 8576
DEG_R = 8704
DEG_OUT = N_CHUNKS * DEG_R
BATCH = 128
SECT = 4000


@functools.lru_cache(maxsize=None)
def _build_sc_agg(n_src, n_edges, with_deg):
    stripe = n_edges // 16
    n_sect = stripe // SECT
    cap = SECT + 160
    dump = cap - LANES
    mesh = plsc.VectorSubcoreMesh(core_axis_name="c", subcore_axis_name="s",
                                  num_cores=2, num_subcores=16)

    out_type = [jax.ShapeDtypeStruct((N_PAD, D), jnp.float32)]
    if with_deg:
        out_type.append(jax.ShapeDtypeStruct((DEG_OUT,), jnp.float32))

    scratch = [
        pltpu.VMEM((stripe,), jnp.int32),
        pltpu.VMEM((stripe,), jnp.int32),
        pltpu.VMEM((cap,), jnp.int32),
        pltpu.VMEM((cap,), jnp.int32),
        pltpu.VMEM((1, BATCH // 2), jnp.int32),
        pltpu.VMEM((1, BATCH // 2), jnp.int32),
        pltpu.VMEM((BATCH, D), jnp.float32),
        pltpu.VMEM_SHARED((ACC_R, D), jnp.float32),
        pltpu.SemaphoreType.DMA,
        pltpu.SemaphoreType.DMA,
        pltpu.SemaphoreType.DMA,
        pltpu.SemaphoreType.DMA,
    ]
    if with_deg:
        scratch += [
            pltpu.VMEM((BATCH,), jnp.float32),
            pltpu.VMEM((544,), jnp.float32),
            pltpu.VMEM((544,), jnp.float32),
            pltpu.VMEM_SHARED((DEG_R,), jnp.float32),
        ]

    def body(x_hbm, src_hbm, dst_hbm, out_hbm, *rest):
        if with_deg:
            (deg_hbm, sstage, dstage, gflat, lflat, curla, curlb, rows,
             acc, gsa, gsb, ssa, ssb, ones_v, zdeg, dstg, dacc) = rest
        else:
            (sstage, dstage, gflat, lflat, curla, curlb, rows,
             acc, gsa, gsb, ssa, ssb) = rest
        half = BATCH // 2
        cid = lax.axis_index("c")
        sid = lax.axis_index("s")
        zv = jnp.zeros((LANES,), jnp.float32)
        lane = lax.iota(jnp.int32, LANES)
        zvi = jnp.zeros((LANES,), jnp.int32)
        tvi = jnp.full((LANES,), S_CHUNK, jnp.int32)

        if with_deg:
            ov = jnp.ones((LANES,), jnp.float32)
            for k in range(BATCH // LANES):
                ones_v[pl.ds(k * LANES, LANES)] = ov

            def zd(i, _):
                zdeg[pl.ds(i * LANES, LANES)] = zv
                return 0
            lax.fori_loop(0, 544 // LANES, zd, 0)

        def zero_rows(i, _):
            rows[i // 8, pl.ds((i % 8) * LANES, LANES)] = zv
            return 0

        pltpu.sync_copy(src_hbm.at[pl.ds(sid * stripe, stripe)], sstage)
        pltpu.sync_copy(dst_hbm.at[pl.ds(sid * stripe, stripe)], dstage)

        def drain_scatters():
            pltpu.make_async_copy(rows.at[pl.ds(0, half)],
                                  acc.at[curla.at[0]], ssa).wait()
            pltpu.make_async_copy(rows.at[pl.ds(half, half)],
                                  acc.at[curlb.at[0]], ssb).wait()
            if with_deg:
                pltpu.make_async_copy(ones_v.at[pl.ds(0, half)],
                                      dacc.at[curla.at[0]], ssa).wait()
                pltpu.make_async_copy(ones_v.at[pl.ds(0, half)],
                                      dacc.at[curlb.at[0]], ssb).wait()

        def issue_scatters():
            pltpu.async_copy(rows.at[pl.ds(0, half)], acc.at[curla.at[0]],
                             ssa, add=True)
            pltpu.async_copy(rows.at[pl.ds(half, half)], acc.at[curlb.at[0]],
                             ssb, add=True)
            if with_deg:
                pltpu.async_copy(ones_v.at[pl.ds(0, half)],
                                 dacc.at[curla.at[0]], ssa, add=True)
                pltpu.async_copy(ones_v.at[pl.ds(0, half)],
                                 dacc.at[curlb.at[0]], ssb, add=True)

        def fire(j, _):
            drain_scatters()
            ga = pltpu.async_copy(
                x_hbm.at[gflat.at[pl.ds(j * BATCH, half)]],
                rows.at[pl.ds(0, half)], gsa)
            gb = pltpu.async_copy(
                x_hbm.at[gflat.at[pl.ds(j * BATCH + half, half)]],
                rows.at[pl.ds(half, half)], gsb)
            for k in range(half // LANES):
                curla[0, pl.ds(k * LANES, LANES)] = lflat[pl.ds(j * BATCH + k * LANES, LANES)]
                curlb[0, pl.ds(k * LANES, LANES)] = lflat[pl.ds(j * BATCH + half + k * LANES, LANES)]
            ga.wait()
            gb.wait()
            issue_scatters()
            return 0

        for c_local in range(N_CHUNKS // 2):
            chunk = cid * (N_CHUNKS // 2) + c_local
            lo = chunk * S_CHUNK

            lax.fori_loop(0, BATCH * 8, zero_rows, 0)
            r0 = sid * (ACC_R // 16)
            for off, ln in ((0, 128), (128, 128), (256, 128), (384, 128),
                            (512, 24)):
                pltpu.sync_copy(rows.at[pl.ds(0, ln)], acc.at[pl.ds(r0 + off, ln)])
            if with_deg:
                pltpu.sync_copy(zdeg, dacc.at[pl.ds(sid * 544, 544)])
            plsc.subcore_barrier()
            for k in range(half // LANES):
                curla[0, pl.ds(k * LANES, LANES)] = tvi
                curlb[0, pl.ds(k * LANES, LANES)] = tvi
            issue_scatters()

            def section(s, f):
                base = s * SECT

                def comp(i, fc):
                    o = base + i * (2 * LANES)
                    sv1 = sstage[pl.ds(o, LANES)]
                    dv1 = dstage[pl.ds(o, LANES)]
                    sv2 = sstage[pl.ds(o + LANES, LANES)]
                    dv2 = dstage[pl.ds(o + LANES, LANES)]
                    dl1 = dv1 - lo
                    dl2 = dv2 - lo
                    m1 = (dl1 >= 0) & (dl1 < S_CHUNK)
                    m2 = (dl2 >= 0) & (dl2 < S_CHUNK)
                    mi1 = m1.astype(jnp.int32)
                    mi2 = m2.astype(jnp.int32)
                    ex1 = jnp.cumsum(mi1) - mi1
                    ex2 = jnp.cumsum(mi2) - mi2
                    s1 = jnp.sum(mi1)
                    pos1 = jnp.where(m1, fc + ex1, dump + lane)
                    pos2 = jnp.where(m2, fc + s1 + ex2, dump + lane)
                    plsc.store_scatter(gflat, [pos1], sv1)
                    plsc.store_scatter(lflat, [pos1], dl1)
                    plsc.store_scatter(gflat, [pos2], sv2)
                    plsc.store_scatter(lflat, [pos2], dl2)
                    return fc + s1 + jnp.sum(mi2)
                f = lax.fori_loop(0, SECT // (2 * LANES), comp, f)
                nbf = f // BATCH
                lax.fori_loop(0, nbf, fire, 0)
                for k in range(BATCH // LANES):
                    gv = gflat[pl.ds(nbf * BATCH + k * LANES, LANES)]
                    lv = lflat[pl.ds(nbf * BATCH + k * LANES, LANES)]
                    gflat[pl.ds(k * LANES, LANES)] = gv
                    lflat[pl.ds(k * LANES, LANES)] = lv
                return f - nbf * BATCH
            f = lax.fori_loop(0, n_sect, section, jnp.int32(0))

            def padb(i, _):
                off = f + i * LANES
                gflat[pl.ds(off, LANES)] = zvi
                lflat[pl.ds(off, LANES)] = tvi
                return 0
            lax.fori_loop(0, (BATCH - f + LANES - 1) // LANES, padb, 0)
            lax.fori_loop(0, (f + BATCH - 1) // BATCH, fire, 0)
            drain_scatters()
            plsc.subcore_barrier()

            e0 = sid * (S_CHUNK // 16)
            pltpu.sync_copy(acc.at[pl.ds(e0, S_CHUNK // 16)],
                            out_hbm.at[pl.ds(lo + e0, S_CHUNK // 16)])
            if with_deg:
                pltpu.sync_copy(dacc.at[pl.ds(sid * 544, 544)], dstg)
                pltpu.sync_copy(dstg,
                                deg_hbm.at[pl.ds(chunk * DEG_R + sid * 544, 544)])
            plsc.subcore_barrier()

    return pl.kernel(
        body, out_type=out_type, mesh=mesh, scratch_types=scratch,
        compiler_params=pltpu.CompilerParams(needs_layout_passes=False))


def _sc_agg(x, src, dst, with_deg):
    fn = _build_sc_agg(x.shape[0], src.shape[0], with_deg)
    out = fn(x, src, dst)
    return out if with_deg else out[0]


def _dense(x, agg, deg, Ws, Wn, b, relu):
    n = x.shape[0]
    blk = 400

    def body(x_ref, a_ref, d_ref, ws_ref, wn_ref, b_ref, o_ref):
        inv = 1.0 / jnp.maximum(d_ref[...], 1.0)
        h = a_ref[...] * inv
        acc = jnp.dot(x_ref[...], ws_ref[...], preferred_element_type=jnp.float32)
        acc = acc + jnp.dot(h, wn_ref[...], preferred_element_type=jnp.float32)
        acc = acc + b_ref[...]
        if relu:
            acc = jnp.maximum(acc, 0.0)
        o_ref[...] = acc

    return pl.pallas_call(
        body,
        grid=(n // blk,),
        in_specs=[
            pl.BlockSpec((blk, D), lambda i: (i, 0)),
            pl.BlockSpec((blk, D), lambda i: (i, 0)),
            pl.BlockSpec((blk, 1), lambda i: (i, 0)),
            pl.BlockSpec((D, D), lambda i: (0, 0)),
            pl.BlockSpec((D, D), lambda i: (0, 0)),
            pl.BlockSpec((1, D), lambda i: (0, 0)),
        ],
        out_specs=pl.BlockSpec((blk, D), lambda i: (i, 0)),
        out_shape=jax.ShapeDtypeStruct((n, D), jnp.float32),
    )(x, agg, deg, Ws, Wn, b.reshape(1, D))


def kernel(x_user, x_item, edge_index_clicks, edge_index_clicked_by,
           Wn0_c, Ws0_c, b0_c, Wn0_cb, Ws0_cb, b0_cb,
           Wn1_c, Ws1_c, b1_c, Wn1_cb, Ws1_cb, b1_cb):
    sc = edge_index_clicks[0].astype(jnp.int32)
    dc = edge_index_clicks[1].astype(jnp.int32)
    scb = edge_index_clicked_by[0].astype(jnp.int32)
    dcb = edge_index_clicked_by[1].astype(jnp.int32)

    agg0_c, deg_c_raw = _sc_agg(x_user, sc, dc, True)
    agg0_cb, deg_cb_raw = _sc_agg(x_item, scb, dcb, True)
    deg_c = deg_c_raw.reshape(N_CHUNKS, DEG_R)[:, :S_CHUNK].reshape(N_PAD, 1)
    deg_cb = deg_cb_raw.reshape(N_CHUNKS, DEG_R)[:, :S_CHUNK].reshape(N_PAD, 1)

    h_item = _dense(x_item, agg0_c, deg_c, Ws0_c, Wn0_c, b0_c, True)
    h_user = _dense(x_user, agg0_cb, deg_cb, Ws0_cb, Wn0_cb, b0_cb, True)

    agg1_c = _sc_agg(h_user, sc, dc, False)
    agg1_cb = _sc_agg(h_item, scb, dcb, False)

    out_item = _dense(h_item, agg1_c, deg_c, Ws1_c, Wn1_c, b1_c, False)
    out_user = _dense(h_user, agg1_cb, deg_cb, Ws1_cb, Wn1_cb, b1_cb, False)
    return (out_user, out_item)
